# baseline (speedup 1.0000x reference)
import functools

import jax
import jax.numpy as jnp
from jax import lax
from jax.experimental import pallas as pl
from jax.experimental.pallas import tpu as pltpu
from jax.experimental.pallas import tpu_sc as plsc

N_ROWS = 16384 * 50
D_EMBED = 64
CHUNK = 128
NBUF = 8
GDEPTH = 4


def _build_gather():
    info = plsc.get_sparse_core_info()
    num_cores, num_subcores = info.num_cores, info.num_subcores
    num_workers = num_cores * num_subcores
    rows_per_worker = N_ROWS // num_workers
    chunks_per_worker = rows_per_worker // CHUNK

    mesh = plsc.VectorSubcoreMesh(core_axis_name="c", subcore_axis_name="s")

    @functools.partial(
        pl.kernel,
        mesh=mesh,
        out_type=jax.ShapeDtypeStruct((N_ROWS, D_EMBED), jnp.float32),
        compiler_params=pltpu.CompilerParams(use_tc_tiling_on_sc=False),
        scratch_types=[
            pltpu.VMEM((chunks_per_worker, CHUNK), jnp.int32),
            pltpu.VMEM((NBUF, CHUNK, D_EMBED), jnp.float32),
            pltpu.SemaphoreType.DMA((NBUF,)),
            pltpu.SemaphoreType.DMA((NBUF,)),
        ],
    )
    def gather_kernel(table_hbm, idx_hbm, out_hbm, idx_v, rows, gsem, ssem):
        wid = lax.axis_index("s") * num_cores + lax.axis_index("c")
        base_chunk = wid * chunks_per_worker
        base_row = wid * rows_per_worker

        pltpu.sync_copy(idx_hbm.at[pl.ds(base_chunk, chunks_per_worker)], idx_v)

        def fire_gather(j, b):
            pltpu.async_copy(table_hbm.at[idx_v.at[j]], rows.at[b], gsem.at[b])

        def wait_gather(j, b):
            pltpu.make_async_copy(
                table_hbm.at[idx_v.at[j]], rows.at[b], gsem.at[b]
            ).wait()

        def out_slice(j):
            return out_hbm.at[pl.ds(base_row + j * CHUNK, CHUNK)]

        def fire_store(j, b):
            pltpu.async_copy(rows.at[b], out_slice(j), ssem.at[b])

        def wait_store(b):
            pltpu.make_async_copy(rows.at[b], out_slice(0), ssem.at[b]).wait()

        for j in range(GDEPTH):
            fire_gather(j, j % NBUF)
        for j in range(GDEPTH, NBUF):
            wait_gather(j - GDEPTH, (j - GDEPTH) % NBUF)
            fire_store(j - GDEPTH, (j - GDEPTH) % NBUF)
            fire_gather(j, j % NBUF)

        def body(g, carry):
            for b in range(NBUF):
                j = g * NBUF + b
                wait_store(b)
                fire_gather(j, b)
                bm = (b - GDEPTH) % NBUF
                wait_gather(j - GDEPTH, bm)
                fire_store(j - GDEPTH, bm)
            return carry

        lax.fori_loop(1, chunks_per_worker // NBUF, body, 0)

        last = chunks_per_worker
        for j in range(last - GDEPTH, last):
            wait_gather(j, j % NBUF)
            fire_store(j, j % NBUF)
        for b in range(NBUF):
            wait_store(b)

    return gather_kernel


_gather = _build_gather()


def kernel(pathway_indices, embedding_table):
    idx2d = pathway_indices.reshape(N_ROWS // CHUNK, CHUNK).astype(jnp.int32)
    flat = _gather(embedding_table, idx2d)
    return flat.reshape(*pathway_indices.shape, D_EMBED)
